# Initial kernel scaffold; baseline (speedup 1.0000x reference)
import jax
import jax.numpy as jnp
from jax.experimental import pallas as pl


def kernel(x, token_table, pos_table):
    raise NotImplementedError("write your pallas kernel here")


# speedup vs baseline: 2.3978x; 2.3978x over previous
import functools

import jax
import jax.numpy as jnp
from jax import lax
from jax.experimental import pallas as pl
from jax.experimental.pallas import tpu as pltpu
from jax.experimental.pallas import tpu_sc as plsc

_LANES = 16
_CHUNK = 100


@functools.lru_cache(maxsize=None)
def _build(B, S, E, V):
    info = plsc.get_sparse_core_info()
    nw = info.num_cores * info.num_subcores
    assert B % nw == 0, (B, nw)
    assert S % _CHUNK == 0 and E % _LANES == 0
    rows_per_w = B // nw
    n_chunks = S // _CHUNK
    e_vecs = E // _LANES

    mesh = plsc.VectorSubcoreMesh(core_axis_name="c", subcore_axis_name="s")

    @functools.partial(
        pl.kernel,
        mesh=mesh,
        out_type=jax.ShapeDtypeStruct((B, n_chunks, _CHUNK, E), jnp.float32),
        scratch_types=[
            pltpu.VMEM((n_chunks, _CHUNK), jnp.int32),
            pltpu.VMEM((n_chunks, _CHUNK, E), jnp.float32),
            pltpu.VMEM((n_chunks, _CHUNK, E), jnp.float32),
            pltpu.SemaphoreType.DMA,
        ],
        compiler_params=pltpu.CompilerParams(use_tc_tiling_on_sc=False),
    )
    def k(x_hbm, tok_hbm, pos_hbm, out_hbm, idx_v, rows_v, pos_v, sem):
        wid = lax.axis_index("s") * info.num_cores + lax.axis_index("c")
        pltpu.sync_copy(pos_hbm, pos_v)

        def body(i, _):
            b = wid * rows_per_w + i
            pltpu.sync_copy(x_hbm.at[b], idx_v)
            cps = [
                pltpu.async_copy(tok_hbm.at[idx_v.at[c]], rows_v.at[c], sem)
                for c in range(n_chunks)
            ]
            for cp in cps:
                cp.wait()

            def add_pos(s, _):
                for c in range(n_chunks):
                    for j in range(e_vecs):
                        sl = pl.ds(j * _LANES, _LANES)
                        rows_v[c, s, sl] = rows_v[c, s, sl] + pos_v[c, s, sl]
                return 0

            lax.fori_loop(0, _CHUNK, add_pos, 0)
            pltpu.sync_copy(rows_v, out_hbm.at[b])
            return 0

        lax.fori_loop(0, rows_per_w, body, 0)

    return k


def kernel(x, token_table, pos_table):
    B, S = x.shape
    V, E = token_table.shape
    k = _build(B, S, E, V)
    xi = x.astype(jnp.int32).reshape(B, S // _CHUNK, _CHUNK)
    pos = pos_table.reshape(S // _CHUNK, _CHUNK, E)
    out = k(xi, token_table, pos)
    return out.reshape(B, S, E)


# speedup vs baseline: 2.8465x; 1.1871x over previous
import functools

import jax
import jax.numpy as jnp
from jax import lax
from jax.experimental import pallas as pl
from jax.experimental.pallas import tpu as pltpu
from jax.experimental.pallas import tpu_sc as plsc

_LANES = 16
_CHUNK = 100


@functools.lru_cache(maxsize=None)
def _build(B, S, E, V):
    info = plsc.get_sparse_core_info()
    nw = info.num_cores * info.num_subcores
    assert B % nw == 0, (B, nw)
    assert S % _CHUNK == 0 and E % _LANES == 0
    rpw = B // nw
    assert rpw >= 4 and rpw % 2 == 0
    n_chunks = S // _CHUNK
    e_vecs = E // _LANES

    mesh = plsc.VectorSubcoreMesh(core_axis_name="c", subcore_axis_name="s")

    @functools.partial(
        pl.kernel,
        mesh=mesh,
        out_type=jax.ShapeDtypeStruct((B, n_chunks, _CHUNK, E), jnp.float32),
        scratch_types=[
            pltpu.VMEM((rpw, n_chunks, _CHUNK), jnp.int32),
            pltpu.VMEM((2, n_chunks, _CHUNK, E), jnp.float32),
            pltpu.VMEM((n_chunks, _CHUNK, E), jnp.float32),
            pltpu.SemaphoreType.DMA,
            pltpu.SemaphoreType.DMA,
            pltpu.SemaphoreType.DMA,
            pltpu.SemaphoreType.DMA,
        ],
        compiler_params=pltpu.CompilerParams(use_tc_tiling_on_sc=False),
    )
    def k(x_hbm, tok_hbm, pos_hbm, out_hbm, idx_v, rows_v, pos_v,
          sg0, sg1, ss0, ss1):
        wid = lax.axis_index("s") * info.num_cores + lax.axis_index("c")
        base = wid * rpw
        sem_g = (sg0, sg1)
        sem_s = (ss0, ss1)

        pltpu.sync_copy(x_hbm.at[wid], idx_v)
        pltpu.sync_copy(pos_hbm, pos_v)

        def fetch(i, u):
            for c in range(n_chunks):
                pltpu.async_copy(
                    tok_hbm.at[idx_v.at[i].at[c]], rows_v.at[u].at[c], sem_g[u])

        def wait_g(u):
            pltpu.make_async_copy(out_hbm.at[0], rows_v.at[u], sem_g[u]).wait()

        def store(i, u):
            pltpu.async_copy(rows_v.at[u], out_hbm.at[base + i], sem_s[u])

        def wait_s(u):
            pltpu.make_async_copy(out_hbm.at[0], rows_v.at[u], sem_s[u]).wait()

        def add_pos(u):
            def body(s, _):
                for c in range(n_chunks):
                    for j in range(e_vecs):
                        sl = pl.ds(j * _LANES, _LANES)
                        rows_v[u, c, s, sl] = rows_v[u, c, s, sl] + pos_v[c, s, sl]
                return 0
            lax.fori_loop(0, _CHUNK, body, 0)

        fetch(0, 0)
        wait_g(0)
        fetch(1, 1)
        add_pos(0)
        store(0, 0)

        def group(g, _):
            for u in (0, 1):
                i = 1 + 2 * g + u
                cur = (1 + u) % 2
                oth = 1 - cur
                wait_g(cur)
                wait_s(oth)
                fetch(i + 1, oth)
                add_pos(cur)
                store(i, cur)
            return 0

        lax.fori_loop(0, (rpw - 2) // 2, group, 0)

        wait_g(1)
        wait_s(0)
        add_pos(1)
        store(rpw - 1, 1)
        wait_s(1)

    return k


def kernel(x, token_table, pos_table):
    B, S = x.shape
    V, E = token_table.shape
    k = _build(B, S, E, V)
    info = plsc.get_sparse_core_info()
    nw = info.num_cores * info.num_subcores
    xi = x.astype(jnp.int32).reshape(nw, B // nw, S // _CHUNK, _CHUNK)
    pos = pos_table.reshape(S // _CHUNK, _CHUNK, E)
    out = k(xi, token_table, pos)
    return out.reshape(B, S, E)


# speedup vs baseline: 3.1383x; 1.1025x over previous
import functools

import jax
import jax.numpy as jnp
from jax import lax
from jax.experimental import pallas as pl
from jax.experimental.pallas import tpu as pltpu
from jax.experimental.pallas import tpu_sc as plsc

_LANES = 16


@functools.lru_cache(maxsize=None)
def _build(B, S, E, V):
    info = plsc.get_sparse_core_info()
    nw = info.num_cores * info.num_subcores
    assert B % nw == 0, (B, nw)
    assert E % _LANES == 0
    rpw = B // nw
    assert rpw >= 4 and rpw % 2 == 0
    e_vecs = E // _LANES
    chunks = []
    off = 0
    while off < S:
        sz = min(128, S - off)
        chunks.append((off, sz))
        off += sz

    mesh = plsc.VectorSubcoreMesh(core_axis_name="c", subcore_axis_name="s")

    @functools.partial(
        pl.kernel,
        mesh=mesh,
        out_type=jax.ShapeDtypeStruct((B, S, E), jnp.float32),
        scratch_types=[
            pltpu.VMEM((rpw, S), jnp.int32),
            pltpu.VMEM((2, S, E), jnp.float32),
            pltpu.VMEM((S, E), jnp.float32),
            pltpu.SemaphoreType.DMA,
            pltpu.SemaphoreType.DMA,
            pltpu.SemaphoreType.DMA,
            pltpu.SemaphoreType.DMA,
        ],
        compiler_params=pltpu.CompilerParams(use_tc_tiling_on_sc=False),
    )
    def k(x_hbm, tok_hbm, pos_hbm, out_hbm, idx_v, rows_v, pos_v,
          sg0, sg1, ss0, ss1):
        wid = lax.axis_index("s") * info.num_cores + lax.axis_index("c")
        base = wid * rpw
        sem_g = (sg0, sg1)
        sem_s = (ss0, ss1)

        pltpu.sync_copy(x_hbm.at[pl.ds(base, rpw)], idx_v)
        pltpu.sync_copy(pos_hbm, pos_v)

        def fetch(i, u):
            for off, sz in chunks:
                pltpu.async_copy(
                    tok_hbm.at[idx_v.at[i].at[pl.ds(off, sz)]],
                    rows_v.at[u].at[pl.ds(off, sz)],
                    sem_g[u])

        def wait_g(u):
            pltpu.make_async_copy(out_hbm.at[0], rows_v.at[u], sem_g[u]).wait()

        def store(i, u):
            pltpu.async_copy(rows_v.at[u], out_hbm.at[base + i], sem_s[u])

        def wait_s(u):
            pltpu.make_async_copy(out_hbm.at[0], rows_v.at[u], sem_s[u]).wait()

        def add_pos(u):
            def body(s, _):
                for j in range(e_vecs):
                    sl = pl.ds(j * _LANES, _LANES)
                    rows_v[u, s, sl] = rows_v[u, s, sl] + pos_v[s, sl]
                return 0
            lax.fori_loop(0, S, body, 0)

        fetch(0, 0)
        wait_g(0)
        fetch(1, 1)
        add_pos(0)
        store(0, 0)

        def group(g, _):
            for u in (0, 1):
                i = 1 + 2 * g + u
                cur = (1 + u) % 2
                oth = 1 - cur
                wait_g(cur)
                wait_s(oth)
                fetch(i + 1, oth)
                add_pos(cur)
                store(i, cur)
            return 0

        lax.fori_loop(0, (rpw - 2) // 2, group, 0)

        wait_g(1)
        wait_s(0)
        add_pos(1)
        store(rpw - 1, 1)
        wait_s(1)

    return k


def kernel(x, token_table, pos_table):
    B, S = x.shape
    V, E = token_table.shape
    k = _build(B, S, E, V)
    return k(x.astype(jnp.int32), token_table, pos_table)


# speedup vs baseline: 3.6585x; 1.1658x over previous
import functools

import jax
import jax.numpy as jnp
from jax import lax
from jax.experimental import pallas as pl
from jax.experimental.pallas import tpu as pltpu
from jax.experimental.pallas import tpu_sc as plsc

_LANES = 16


@functools.lru_cache(maxsize=None)
def _build(B, S, E, V):
    info = plsc.get_sparse_core_info()
    nw = info.num_cores * info.num_subcores
    assert B % nw == 0, (B, nw)
    assert E % _LANES == 0 and S % 2 == 0
    rpw = B // nw
    assert rpw >= 6 and rpw % 2 == 0
    e_vecs = E // _LANES
    s2 = S // 2
    wide = 2 * E
    chunks = []
    off = 0
    while off < S:
        sz = min(128, S - off)
        chunks.append((off, sz))
        off += sz

    mesh = plsc.VectorSubcoreMesh(core_axis_name="c", subcore_axis_name="s")

    @functools.partial(
        pl.kernel,
        mesh=mesh,
        out_type=jax.ShapeDtypeStruct((B, s2, wide), jnp.float32),
        scratch_types=[
            pltpu.VMEM((rpw, S), jnp.int32),
            pltpu.VMEM((2, S, E), jnp.float32),
            pltpu.VMEM((2, s2, wide), jnp.float32),
            pltpu.VMEM((s2, wide), jnp.float32),
            pltpu.SemaphoreType.DMA,
            pltpu.SemaphoreType.DMA,
            pltpu.SemaphoreType.DMA,
            pltpu.SemaphoreType.DMA,
        ],
        compiler_params=pltpu.CompilerParams(use_tc_tiling_on_sc=False),
    )
    def k(x_hbm, tok_hbm, pos_hbm, out_hbm, idx_v, g_v, rows_v, pos_v,
          sg0, sg1, ss0, ss1):
        wid = lax.axis_index("s") * info.num_cores + lax.axis_index("c")
        base = wid * rpw
        sem_g = (sg0, sg1)
        sem_s = (ss0, ss1)

        pltpu.sync_copy(x_hbm.at[pl.ds(base, rpw)], idx_v)
        pltpu.sync_copy(pos_hbm, pos_v)

        def fetch(i, u):
            for off, sz in chunks:
                pltpu.async_copy(
                    tok_hbm.at[idx_v.at[i].at[pl.ds(off, sz)]],
                    g_v.at[u].at[pl.ds(off, sz)],
                    sem_g[u])

        def wait_g(u):
            pltpu.make_async_copy(
                tok_hbm.at[pl.ds(0, S)], g_v.at[u], sem_g[u]).wait()

        def store(i, u):
            pltpu.async_copy(rows_v.at[u], out_hbm.at[base + i], sem_s[u])

        def wait_s(u):
            pltpu.make_async_copy(out_hbm.at[0], rows_v.at[u], sem_s[u]).wait()

        def add_pos(u):
            def body(p, _):
                for h in (0, 1):
                    for j in range(e_vecs):
                        src = pl.ds(j * _LANES, _LANES)
                        dst = pl.ds(h * E + j * _LANES, _LANES)
                        rows_v[u, p, dst] = g_v[u, 2 * p + h, src] + pos_v[p, dst]
                return 0
            lax.fori_loop(0, s2, body, 0)

        fetch(0, 0)
        wait_g(0)
        fetch(1, 1)
        add_pos(0)
        store(0, 0)

        wait_g(1)
        fetch(2, 0)
        add_pos(1)
        store(1, 1)

        def group(g, _):
            for u in (0, 1):
                i = 2 + 2 * g + u
                cur = u
                oth = 1 - u
                wait_g(cur)
                fetch(i + 1, oth)
                wait_s(cur)
                add_pos(cur)
                store(i, cur)
            return 0

        lax.fori_loop(0, (rpw - 4) // 2, group, 0)

        wait_g(0)
        fetch(rpw - 1, 1)
        wait_s(0)
        add_pos(0)
        store(rpw - 2, 0)

        wait_g(1)
        wait_s(1)
        add_pos(1)
        store(rpw - 1, 1)

        wait_s(0)
        wait_s(1)

    return k


def kernel(x, token_table, pos_table):
    B, S = x.shape
    V, E = token_table.shape
    k = _build(B, S, E, V)
    pos2 = pos_table.reshape(S // 2, 2 * E)
    out = k(x.astype(jnp.int32), token_table, pos2)
    return out.reshape(B, S, E)
